# baseline (speedup 1.0000x reference)
import functools

import jax
import jax.numpy as jnp
from jax import lax
from jax.experimental import pallas as pl
from jax.experimental.pallas import tpu as pltpu
from jax.experimental.pallas import tpu_sc as plsc

N = 10000
E = 160000
F = 256
EF = 49
OC = 20
Z = 200

NP_ = 10240
KC = EF + 1
UW = 1024
OCP = 32
NC, NS = 2, 16
NW = NC * NS
CH = 32
NCHUNK = 158
PER_W = CH * NCHUNK
EP = NW * PER_W
PAD_DST = N + 100
ZROWS = NP_ // NS
RB = 1024
GR = NP_ // RB


def _mm_u(x_ref, w_ref, o_ref):
    o_ref[...] = jnp.dot(x_ref[...], w_ref[...],
                         preferred_element_type=jnp.float32
                         ).astype(jnp.bfloat16)


def _tc_u(h_pad, mflat):
    return pl.pallas_call(
        _mm_u,
        grid=(GR,),
        in_specs=[
            pl.BlockSpec((RB, F), lambda i: (i, 0)),
            pl.BlockSpec((F, UW), lambda i: (0, 0)),
        ],
        out_specs=pl.BlockSpec((RB, UW), lambda i: (i, 0)),
        out_shape=jax.ShapeDtypeStruct((NP_, UW), jnp.bfloat16),
    )(h_pad, mflat)


def _mm_upd(p_ref, h_ref, w1a_ref, w1b_ref, ba1_ref, wa2_ref, ba2_ref, o_ref):
    agg = p_ref[0] + p_ref[1]
    y = jnp.dot(agg, w1a_ref[...], preferred_element_type=jnp.float32)
    y = y + jnp.dot(h_ref[...], w1b_ref[...], preferred_element_type=jnp.float32)
    y = jnp.maximum(y + ba1_ref[...], 0.0)
    o_ref[...] = jnp.dot(y, wa2_ref[...],
                         preferred_element_type=jnp.float32) + ba2_ref[...]


def _tc_update(partials, h_pad, w1a, w1b, ba1, wa2, ba2):
    return pl.pallas_call(
        _mm_upd,
        grid=(GR,),
        in_specs=[
            pl.BlockSpec((2, RB, OCP), lambda i: (0, i, 0)),
            pl.BlockSpec((RB, F), lambda i: (i, 0)),
            pl.BlockSpec((OCP, F), lambda i: (0, 0)),
            pl.BlockSpec((F, F), lambda i: (0, 0)),
            pl.BlockSpec((1, F), lambda i: (0, 0)),
            pl.BlockSpec((F, F), lambda i: (0, 0)),
            pl.BlockSpec((1, F), lambda i: (0, 0)),
        ],
        out_specs=pl.BlockSpec((RB, F), lambda i: (i, 0)),
        out_shape=jax.ShapeDtypeStruct((NP_, F), jnp.float32),
    )(partials, h_pad, w1a, w1b, ba1, wa2, ba2)


def _mm_heads(h_ref, w_ref, b_ref, o_ref):
    o_ref[...] = jnp.dot(h_ref[...], w_ref[...],
                         preferred_element_type=jnp.float32) + b_ref[...]


def _tc_heads(h_pad, whead, bhead):
    return pl.pallas_call(
        _mm_heads,
        grid=(GR,),
        in_specs=[
            pl.BlockSpec((RB, F), lambda i: (i, 0)),
            pl.BlockSpec((F, 512), lambda i: (0, 0)),
            pl.BlockSpec((1, 512), lambda i: (0, 0)),
        ],
        out_specs=pl.BlockSpec((RB, 512), lambda i: (i, 0)),
        out_shape=jax.ShapeDtypeStruct((NP_, 512), jnp.float32),
    )(h_pad, whead, bhead)


def _sc_edge_body(u_hbm, src_hbm, dst_hbm, a_hbm, out_hbm,
                  srcall, dstall, av, uv, mv, zbuf, aggr,
                  sem_u, sem_a, sem_s):
    c = lax.axis_index("c")
    s = lax.axis_index("s")
    wid = c * NS + s
    iota16 = lax.iota(jnp.int32, 16)
    zero16 = jnp.zeros((16,), jnp.float32)

    pltpu.sync_copy(src_hbm.at[wid], srcall)
    pltpu.sync_copy(dst_hbm.at[wid], dstall)

    def _zb(i, _):
        n = i * 16 + iota16
        plsc.store_scatter(zbuf, [n // OCP, n % OCP], zero16)
        return 0
    lax.fori_loop(0, (ZROWS * OCP) // 16, _zb, 0)
    pltpu.sync_copy(zbuf, aggr.at[pl.ds(s * ZROWS, ZROWS)])
    plsc.subcore_barrier()

    ebase = wid * PER_W

    def _stage(ci, b):
        e0a = lax.min(ebase + ci * CH, E - CH)
        pltpu.async_copy(a_hbm.at[pl.ds(e0a, CH)], av[b], sem_a[b])
        pltpu.async_copy(u_hbm.at[srcall.at[ci]], uv[b], sem_u[b])

    def _compute(ci, b):
        pltpu.make_async_copy(u_hbm.at[srcall.at[ci]], uv[b], sem_u[b]).wait()
        pltpu.make_async_copy(a_hbm.at[pl.ds(ebase, CH)], av[b],
                              sem_a[b]).wait()

        @pl.when(ci >= 2)
        def _():
            pltpu.make_async_copy(mv[b], aggr.at[dstall.at[ci]],
                                  sem_s[b]).wait()

        def _unpk(w):
            return plsc.unpack(plsc.bitcast(w, jnp.bfloat16),
                               format=plsc.PackFormat.INTERLEAVED)

        def _edge4(i, _):
            rows = tuple(jnp.full((16,), i * 4 + j, jnp.int32)
                         for j in range(4))
            acce = []
            acco = []
            for j in range(4):
                be, bo = _unpk(plsc.load_gather(
                    uv[b], [rows[j], EF * OC // 2 + iota16]))
                acce.append(be)
                acco.append(bo)

            def _k(k, carry):
                acce, acco, uwc, ac = carry
                for j in range(4):
                    a_k = plsc.load_gather(av[b], [rows[j], ac])
                    ue, uo = _unpk(plsc.load_gather(uv[b], [rows[j], uwc]))
                    acce[j] = acce[j] + a_k * ue
                    acco[j] = acco[j] + a_k * uo
                return acce, acco, uwc + OC // 2, ac + 1

            acce, acco, _, _ = lax.fori_loop(
                0, EF, _k,
                (acce, acco, iota16, jnp.zeros((16,), jnp.int32)))
            for j in range(4):
                me = 1.0 / (1.0 + jnp.exp(-acce[j]))
                mo = 1.0 / (1.0 + jnp.exp(-acco[j]))
                plsc.store_scatter(mv[b], [rows[j], 2 * iota16], me)
                plsc.store_scatter(mv[b], [rows[j], 2 * iota16 + 1], mo)
            return 0

        lax.fori_loop(0, CH // 4, _edge4, 0)
        pltpu.async_copy(mv[b], aggr.at[dstall.at[ci]], sem_s[b], add=True)

    _stage(0, 0)
    _stage(1, 1)

    def _pair(j, _):
        i0 = 2 * j
        _compute(i0, 0)

        @pl.when(i0 + 2 < NCHUNK)
        def _():
            _stage(i0 + 2, 0)
        _compute(i0 + 1, 1)

        @pl.when(i0 + 3 < NCHUNK)
        def _():
            _stage(i0 + 3, 1)
        return 0

    lax.fori_loop(0, NCHUNK // 2, _pair, 0)
    pltpu.make_async_copy(mv[0], aggr.at[dstall.at[0]], sem_s[0]).wait()
    pltpu.make_async_copy(mv[1], aggr.at[dstall.at[0]], sem_s[1]).wait()
    plsc.subcore_barrier()
    pltpu.sync_copy(aggr.at[pl.ds(s * ZROWS, ZROWS)],
                    out_hbm.at[pl.ds(c * NP_ + s * ZROWS, ZROWS)])


_sc_edge = functools.partial(
    pl.kernel,
    mesh=plsc.VectorSubcoreMesh(core_axis_name="c", subcore_axis_name="s",
                                num_cores=NC, num_subcores=NS),
    out_type=jax.ShapeDtypeStruct((NC * NP_, OCP), jnp.float32),
    compiler_params=pltpu.CompilerParams(needs_layout_passes=False,
                                         use_tc_tiling_on_sc=False,
                                         disable_bounds_checks=True),
    scratch_types=[
        pltpu.VMEM((NCHUNK, CH), jnp.int32),
        pltpu.VMEM((NCHUNK, CH), jnp.int32),
        [pltpu.VMEM((CH, EF), jnp.float32)] * 2,
        [pltpu.VMEM((CH, UW // 2), jnp.int32)] * 2,
        [pltpu.VMEM((CH, OCP), jnp.float32)] * 2,
        pltpu.VMEM((ZROWS, OCP), jnp.float32),
        pltpu.VMEM_SHARED((NP_, OCP), jnp.float32),
        [pltpu.SemaphoreType.DMA] * 2,
        [pltpu.SemaphoreType.DMA] * 2,
        [pltpu.SemaphoreType.DMA] * 2,
    ],
)(_sc_edge_body)


def _mk_mflat(Wn, bn):
    m = jnp.concatenate(
        [Wn.reshape(EF, F, OC).transpose(1, 0, 2),
         bn.reshape(1, F, OC).transpose(1, 0, 2)], axis=1)
    m = m.reshape(F, KC * OC)
    return jnp.pad(m, ((0, 0), (0, UW - KC * OC)))


def _layer(h_pad, src_p, dst_p, a_p, mflat, w1a, w1b, ba1, wa2, ba2):
    u = _tc_u(h_pad, mflat)
    u_pk = lax.bitcast_convert_type(
        u.reshape(NP_, UW // 2, 2), jnp.int32)
    partials = _sc_edge(u_pk, src_p, dst_p, a_p).reshape(NC, NP_, OCP)
    return _tc_update(partials, h_pad, w1a, w1b, ba1, wa2, ba2)


def kernel(x, edge_ind, edge_attr, Wn1, bn1, Wa11, ba11, Wa21, ba21,
           Wn2, bn2, Wa12, ba12, Wa22, ba22, Wmu, bmu, Wlv, blv):
    f32 = jnp.float32
    h0 = jnp.pad(x, ((0, NP_ - N), (0, 0)))
    src_p = jnp.pad(edge_ind[0], (0, EP - E)).reshape(NW, NCHUNK, CH)
    dst_p = jnp.pad(edge_ind[1], (0, EP - E),
                    constant_values=PAD_DST).reshape(NW, NCHUNK, CH)
    a_p = edge_attr

    mflat1 = _mk_mflat(Wn1, bn1)
    mflat2 = _mk_mflat(Wn2, bn2)
    w1a1 = jnp.pad(Wa11[:OC], ((0, OCP - OC), (0, 0)))
    w1b1 = Wa11[OC:]
    w1a2 = jnp.pad(Wa12[:OC], ((0, OCP - OC), (0, 0)))
    w1b2 = Wa12[OC:]

    h1 = _layer(h0, src_p, dst_p, a_p, mflat1, w1a1, w1b1,
                ba11.reshape(1, F), Wa21, ba21.reshape(1, F))
    h2 = _layer(h1, src_p, dst_p, a_p, mflat2, w1a2, w1b2,
                ba12.reshape(1, F), Wa22, ba22.reshape(1, F))

    whead = jnp.concatenate(
        [jnp.pad(Wmu, ((0, 0), (0, 256 - Z))),
         jnp.pad(Wlv, ((0, 0), (0, 256 - Z)))], axis=1)
    bhead = jnp.concatenate(
        [jnp.pad(bmu, (0, 256 - Z)), jnp.pad(blv, (0, 256 - Z))]
    ).reshape(1, 512)
    heads = _tc_heads(h2, whead, bhead)
    return (heads[:N, :Z], heads[:N, 256:256 + Z])


# speedup vs baseline: 1.2177x; 1.2177x over previous
import functools

import jax
import jax.numpy as jnp
from jax import lax
from jax.experimental import pallas as pl
from jax.experimental.pallas import tpu as pltpu
from jax.experimental.pallas import tpu_sc as plsc

N = 10000
E = 160000
F = 256
EF = 49
OC = 20
Z = 200

NP_ = 10240
KC = EF + 1
UW = 1024
OCP = 32
NC, NS = 2, 16
NW = NC * NS
CH = 32
NCHUNK = 158
PER_W = CH * NCHUNK
EP = NW * PER_W
PAD_DST = N + 100
ZROWS = NP_ // NS
RB = 1024
GR = NP_ // RB


def _mm_u(x_ref, w_ref, o_ref):
    o_ref[...] = jnp.dot(x_ref[...], w_ref[...],
                         preferred_element_type=jnp.float32)


def _tc_u(h_pad, mflat):
    return pl.pallas_call(
        _mm_u,
        grid=(GR,),
        in_specs=[
            pl.BlockSpec((RB, F), lambda i: (i, 0)),
            pl.BlockSpec((F, UW), lambda i: (0, 0)),
        ],
        out_specs=pl.BlockSpec((RB, UW), lambda i: (i, 0)),
        out_shape=jax.ShapeDtypeStruct((NP_, UW), jnp.float32),
    )(h_pad, mflat)


def _mm_upd(p_ref, h_ref, w1a_ref, w1b_ref, ba1_ref, wa2_ref, ba2_ref, o_ref):
    agg = p_ref[0] + p_ref[1]
    y = jnp.dot(agg, w1a_ref[...], preferred_element_type=jnp.float32)
    y = y + jnp.dot(h_ref[...], w1b_ref[...], preferred_element_type=jnp.float32)
    y = jnp.maximum(y + ba1_ref[...], 0.0)
    o_ref[...] = jnp.dot(y, wa2_ref[...],
                         preferred_element_type=jnp.float32) + ba2_ref[...]


def _tc_update(partials, h_pad, w1a, w1b, ba1, wa2, ba2):
    return pl.pallas_call(
        _mm_upd,
        grid=(GR,),
        in_specs=[
            pl.BlockSpec((2, RB, OCP), lambda i: (0, i, 0)),
            pl.BlockSpec((RB, F), lambda i: (i, 0)),
            pl.BlockSpec((OCP, F), lambda i: (0, 0)),
            pl.BlockSpec((F, F), lambda i: (0, 0)),
            pl.BlockSpec((1, F), lambda i: (0, 0)),
            pl.BlockSpec((F, F), lambda i: (0, 0)),
            pl.BlockSpec((1, F), lambda i: (0, 0)),
        ],
        out_specs=pl.BlockSpec((RB, F), lambda i: (i, 0)),
        out_shape=jax.ShapeDtypeStruct((NP_, F), jnp.float32),
    )(partials, h_pad, w1a, w1b, ba1, wa2, ba2)


def _mm_heads(h_ref, w_ref, b_ref, o_ref):
    o_ref[...] = jnp.dot(h_ref[...], w_ref[...],
                         preferred_element_type=jnp.float32) + b_ref[...]


def _tc_heads(h_pad, whead, bhead):
    return pl.pallas_call(
        _mm_heads,
        grid=(GR,),
        in_specs=[
            pl.BlockSpec((RB, F), lambda i: (i, 0)),
            pl.BlockSpec((F, 512), lambda i: (0, 0)),
            pl.BlockSpec((1, 512), lambda i: (0, 0)),
        ],
        out_specs=pl.BlockSpec((RB, 512), lambda i: (i, 0)),
        out_shape=jax.ShapeDtypeStruct((NP_, 512), jnp.float32),
    )(h_pad, whead, bhead)


def _sc_edge_body(u_hbm, src_hbm, dst_hbm, a_hbm, out_hbm,
                  srcall, dstall, av, uv, mv, zbuf, aggr,
                  sem_u, sem_a, sem_s):
    c = lax.axis_index("c")
    s = lax.axis_index("s")
    wid = c * NS + s
    iota16 = lax.iota(jnp.int32, 16)
    zero16 = jnp.zeros((16,), jnp.float32)

    pltpu.sync_copy(src_hbm.at[wid], srcall)
    pltpu.sync_copy(dst_hbm.at[wid], dstall)

    def _zb(i, _):
        n = i * 16 + iota16
        plsc.store_scatter(zbuf, [n // OCP, n % OCP], zero16)
        return 0
    lax.fori_loop(0, (ZROWS * OCP) // 16, _zb, 0)
    pltpu.sync_copy(zbuf, aggr.at[pl.ds(s * ZROWS, ZROWS)])
    plsc.subcore_barrier()

    ebase = wid * PER_W

    def _stage(ci, b):
        e0a = lax.min(ebase + ci * CH, E - CH)
        pltpu.async_copy(a_hbm.at[pl.ds(e0a, CH)], av[b], sem_a[b])
        pltpu.async_copy(u_hbm.at[srcall.at[ci]], uv[b], sem_u[b])

    def _compute(ci, b):
        pltpu.make_async_copy(u_hbm.at[srcall.at[ci]], uv[b], sem_u[b]).wait()
        pltpu.make_async_copy(a_hbm.at[pl.ds(ebase, CH)], av[b],
                              sem_a[b]).wait()

        @pl.when(ci >= 2)
        def _():
            pltpu.make_async_copy(mv[b], aggr.at[dstall.at[ci]],
                                  sem_s[b]).wait()

        def _edge4(i, _):
            rows = tuple(jnp.full((16,), i * 4 + j, jnp.int32)
                         for j in range(4))
            acc0 = [plsc.load_gather(uv[b], [rows[j], EF * OC + iota16])
                    for j in range(4)]
            acc1 = [plsc.load_gather(uv[b], [rows[j], EF * OC + 16 + iota16])
                    for j in range(4)]

            def _k(k, carry):
                acc0, acc1, u0c, u1c, ac = carry
                for j in range(4):
                    a_k = plsc.load_gather(av[b], [rows[j], ac])
                    acc0[j] = acc0[j] + a_k * plsc.load_gather(
                        uv[b], [rows[j], u0c])
                    acc1[j] = acc1[j] + a_k * plsc.load_gather(
                        uv[b], [rows[j], u1c])
                return acc0, acc1, u0c + OC, u1c + OC, ac + 1

            acc0, acc1, _, _, _ = lax.fori_loop(
                0, EF, _k,
                (acc0, acc1, iota16, 16 + iota16,
                 jnp.zeros((16,), jnp.int32)))
            for j in range(4):
                m0 = 1.0 / (1.0 + jnp.exp(-acc0[j]))
                m1 = 1.0 / (1.0 + jnp.exp(-acc1[j]))
                plsc.store_scatter(mv[b], [rows[j], iota16], m0)
                plsc.store_scatter(mv[b], [rows[j], 16 + iota16], m1)
            return 0

        lax.fori_loop(0, CH // 4, _edge4, 0)
        pltpu.async_copy(mv[b], aggr.at[dstall.at[ci]], sem_s[b], add=True)

    _stage(0, 0)
    _stage(1, 1)

    def _pair(j, _):
        i0 = 2 * j
        _compute(i0, 0)

        @pl.when(i0 + 2 < NCHUNK)
        def _():
            _stage(i0 + 2, 0)
        _compute(i0 + 1, 1)

        @pl.when(i0 + 3 < NCHUNK)
        def _():
            _stage(i0 + 3, 1)
        return 0

    lax.fori_loop(0, NCHUNK // 2, _pair, 0)
    pltpu.make_async_copy(mv[0], aggr.at[dstall.at[0]], sem_s[0]).wait()
    pltpu.make_async_copy(mv[1], aggr.at[dstall.at[0]], sem_s[1]).wait()
    plsc.subcore_barrier()
    pltpu.sync_copy(aggr.at[pl.ds(s * ZROWS, ZROWS)],
                    out_hbm.at[pl.ds(c * NP_ + s * ZROWS, ZROWS)])


_sc_edge = functools.partial(
    pl.kernel,
    mesh=plsc.VectorSubcoreMesh(core_axis_name="c", subcore_axis_name="s",
                                num_cores=NC, num_subcores=NS),
    out_type=jax.ShapeDtypeStruct((NC * NP_, OCP), jnp.float32),
    compiler_params=pltpu.CompilerParams(needs_layout_passes=False,
                                         use_tc_tiling_on_sc=False,
                                         disable_bounds_checks=True),
    scratch_types=[
        pltpu.VMEM((NCHUNK, CH), jnp.int32),
        pltpu.VMEM((NCHUNK, CH), jnp.int32),
        [pltpu.VMEM((CH, EF), jnp.float32)] * 2,
        [pltpu.VMEM((CH, UW), jnp.float32)] * 2,
        [pltpu.VMEM((CH, OCP), jnp.float32)] * 2,
        pltpu.VMEM((ZROWS, OCP), jnp.float32),
        pltpu.VMEM_SHARED((NP_, OCP), jnp.float32),
        [pltpu.SemaphoreType.DMA] * 2,
        [pltpu.SemaphoreType.DMA] * 2,
        [pltpu.SemaphoreType.DMA] * 2,
    ],
)(_sc_edge_body)


def _mk_mflat(Wn, bn):
    m = jnp.concatenate(
        [Wn.reshape(EF, F, OC).transpose(1, 0, 2),
         bn.reshape(1, F, OC).transpose(1, 0, 2)], axis=1)
    m = m.reshape(F, KC * OC)
    return jnp.pad(m, ((0, 0), (0, UW - KC * OC)))


def _layer(h_pad, src_p, dst_p, a_p, mflat, w1a, w1b, ba1, wa2, ba2):
    u = _tc_u(h_pad, mflat)
    partials = _sc_edge(u, src_p, dst_p, a_p).reshape(NC, NP_, OCP)
    return _tc_update(partials, h_pad, w1a, w1b, ba1, wa2, ba2)


def kernel(x, edge_ind, edge_attr, Wn1, bn1, Wa11, ba11, Wa21, ba21,
           Wn2, bn2, Wa12, ba12, Wa22, ba22, Wmu, bmu, Wlv, blv):
    f32 = jnp.float32
    h0 = jnp.pad(x, ((0, NP_ - N), (0, 0)))
    src_p = jnp.pad(edge_ind[0], (0, EP - E)).reshape(NW, NCHUNK, CH)
    dst_p = jnp.pad(edge_ind[1], (0, EP - E),
                    constant_values=PAD_DST).reshape(NW, NCHUNK, CH)
    a_p = edge_attr

    mflat1 = _mk_mflat(Wn1, bn1)
    mflat2 = _mk_mflat(Wn2, bn2)
    w1a1 = jnp.pad(Wa11[:OC], ((0, OCP - OC), (0, 0)))
    w1b1 = Wa11[OC:]
    w1a2 = jnp.pad(Wa12[:OC], ((0, OCP - OC), (0, 0)))
    w1b2 = Wa12[OC:]

    h1 = _layer(h0, src_p, dst_p, a_p, mflat1, w1a1, w1b1,
                ba11.reshape(1, F), Wa21, ba21.reshape(1, F))
    h2 = _layer(h1, src_p, dst_p, a_p, mflat2, w1a2, w1b2,
                ba12.reshape(1, F), Wa22, ba22.reshape(1, F))

    whead = jnp.concatenate(
        [jnp.pad(Wmu, ((0, 0), (0, 256 - Z))),
         jnp.pad(Wlv, ((0, 0), (0, 256 - Z)))], axis=1)
    bhead = jnp.concatenate(
        [jnp.pad(bmu, (0, 256 - Z)), jnp.pad(blv, (0, 256 - Z))]
    ).reshape(1, 512)
    heads = _tc_heads(h2, whead, bhead)
    return (heads[:N, :Z], heads[:N, 256:256 + Z])


# speedup vs baseline: 1.2318x; 1.0117x over previous
import functools

import jax
import jax.numpy as jnp
from jax import lax
from jax.experimental import pallas as pl
from jax.experimental.pallas import tpu as pltpu
from jax.experimental.pallas import tpu_sc as plsc

N = 10000
E = 160000
F = 256
EF = 49
OC = 20
Z = 200

NP_ = 10240
KC = EF + 1
UW = 1024
OCP = 32
NC, NS = 2, 16
NW = NC * NS
CH = 32
NCHUNK = 158
PER_W = CH * NCHUNK
EP = NW * PER_W
PAD_DST = N + 100
ZROWS = NP_ // NS
RB = 1024
GR = NP_ // RB


def _mm_u(x_ref, w_ref, o_ref):
    o_ref[...] = jnp.dot(x_ref[...].astype(jnp.bfloat16),
                         w_ref[...].astype(jnp.bfloat16),
                         preferred_element_type=jnp.float32)


def _tc_u(h_pad, mflat):
    return pl.pallas_call(
        _mm_u,
        grid=(GR,),
        in_specs=[
            pl.BlockSpec((RB, F), lambda i: (i, 0)),
            pl.BlockSpec((F, UW), lambda i: (0, 0)),
        ],
        out_specs=pl.BlockSpec((RB, UW), lambda i: (i, 0)),
        out_shape=jax.ShapeDtypeStruct((NP_, UW), jnp.float32),
    )(h_pad, mflat)


def _mm_upd(p_ref, h_ref, w1a_ref, w1b_ref, ba1_ref, wa2_ref, ba2_ref, o_ref):
    agg = p_ref[0] + p_ref[1]
    y = jnp.dot(agg, w1a_ref[...], preferred_element_type=jnp.float32)
    y = y + jnp.dot(h_ref[...], w1b_ref[...], preferred_element_type=jnp.float32)
    y = jnp.maximum(y + ba1_ref[...], 0.0)
    o_ref[...] = jnp.dot(y, wa2_ref[...],
                         preferred_element_type=jnp.float32) + ba2_ref[...]


def _tc_update(partials, h_pad, w1a, w1b, ba1, wa2, ba2):
    return pl.pallas_call(
        _mm_upd,
        grid=(GR,),
        in_specs=[
            pl.BlockSpec((2, RB, OCP), lambda i: (0, i, 0)),
            pl.BlockSpec((RB, F), lambda i: (i, 0)),
            pl.BlockSpec((OCP, F), lambda i: (0, 0)),
            pl.BlockSpec((F, F), lambda i: (0, 0)),
            pl.BlockSpec((1, F), lambda i: (0, 0)),
            pl.BlockSpec((F, F), lambda i: (0, 0)),
            pl.BlockSpec((1, F), lambda i: (0, 0)),
        ],
        out_specs=pl.BlockSpec((RB, F), lambda i: (i, 0)),
        out_shape=jax.ShapeDtypeStruct((NP_, F), jnp.float32),
    )(partials, h_pad, w1a, w1b, ba1, wa2, ba2)


def _mm_heads(h_ref, w_ref, b_ref, o_ref):
    o_ref[...] = jnp.dot(h_ref[...], w_ref[...],
                         preferred_element_type=jnp.float32) + b_ref[...]


def _tc_heads(h_pad, whead, bhead):
    return pl.pallas_call(
        _mm_heads,
        grid=(GR,),
        in_specs=[
            pl.BlockSpec((RB, F), lambda i: (i, 0)),
            pl.BlockSpec((F, 512), lambda i: (0, 0)),
            pl.BlockSpec((1, 512), lambda i: (0, 0)),
        ],
        out_specs=pl.BlockSpec((RB, 512), lambda i: (i, 0)),
        out_shape=jax.ShapeDtypeStruct((NP_, 512), jnp.float32),
    )(h_pad, whead, bhead)


def _sc_edge_body(u_hbm, src_hbm, dst_hbm, a_hbm, out_hbm,
                  srcall, dstall, av, uv, mv, zbuf, aggr,
                  sem_u, sem_a, sem_s):
    c = lax.axis_index("c")
    s = lax.axis_index("s")
    wid = c * NS + s
    iota16 = lax.iota(jnp.int32, 16)
    zero16 = jnp.zeros((16,), jnp.float32)

    for b in range(2):
        for g in range(CH // 16):
            eids0 = iota16 + g * 16
            for o in range(OC, OCP):
                plsc.store_scatter(mv[b],
                                   [eids0, jnp.full((16,), o, jnp.int32)],
                                   zero16)

    pltpu.sync_copy(src_hbm.at[wid], srcall)
    pltpu.sync_copy(dst_hbm.at[wid], dstall)

    def _zb(i, _):
        n = i * 16 + iota16
        plsc.store_scatter(zbuf, [n // OCP, n % OCP], zero16)
        return 0
    lax.fori_loop(0, (ZROWS * OCP) // 16, _zb, 0)
    pltpu.sync_copy(zbuf, aggr.at[pl.ds(s * ZROWS, ZROWS)])
    plsc.subcore_barrier()

    ebase = wid * PER_W

    def _stage(ci, b):
        pltpu.async_copy(a_hbm.at[pl.ds(ebase + ci * CH, CH)], av[b],
                         sem_a[b])
        pltpu.async_copy(u_hbm.at[srcall.at[ci]], uv[b], sem_u[b])

    def _compute(ci, b):
        pltpu.make_async_copy(u_hbm.at[srcall.at[ci]], uv[b], sem_u[b]).wait()
        pltpu.make_async_copy(a_hbm.at[pl.ds(ebase, CH)], av[b],
                              sem_a[b]).wait()

        @pl.when(ci >= 2)
        def _():
            pltpu.make_async_copy(mv[b], aggr.at[dstall.at[ci]],
                                  sem_s[b]).wait()

        def _edge4(i, _):
            rows = tuple(jnp.full((16,), i * 4 + j, jnp.int32)
                         for j in range(4))
            acc0 = [plsc.load_gather(uv[b], [rows[j], EF * OC + iota16])
                    for j in range(4)]
            acc1 = [plsc.load_gather(uv[b], [rows[j], EF * OC + 16 + iota16])
                    for j in range(4)]

            def _k(k, carry):
                acc0, acc1, u0c, u1c, ac = carry
                for j in range(4):
                    a_k = plsc.load_gather(av[b], [rows[j], ac])
                    acc0[j] = acc0[j] + a_k * plsc.load_gather(
                        uv[b], [rows[j], u0c])
                    acc1[j] = acc1[j] + a_k * plsc.load_gather(
                        uv[b], [rows[j], u1c])
                return acc0, acc1, u0c + OC, u1c + OC, ac + 1

            acc0, acc1, _, _, _ = lax.fori_loop(
                0, EF, _k,
                (acc0, acc1, iota16, 16 + iota16,
                 jnp.zeros((16,), jnp.int32)))
            for j in range(4):
                m0 = 1.0 / (1.0 + jnp.exp(-acc0[j]))
                m1 = 1.0 / (1.0 + jnp.exp(-acc1[j]))
                plsc.store_scatter(mv[b], [rows[j], iota16], m0)
                plsc.store_scatter(mv[b], [rows[j], 16 + iota16], m1)
            return 0

        lax.fori_loop(0, CH // 4, _edge4, 0)
        pltpu.async_copy(mv[b], aggr.at[dstall.at[ci]], sem_s[b], add=True)

    _stage(0, 0)
    _stage(1, 1)

    def _pair(j, _):
        i0 = 2 * j
        _compute(i0, 0)

        @pl.when(i0 + 2 < NCHUNK)
        def _():
            _stage(i0 + 2, 0)
        _compute(i0 + 1, 1)

        @pl.when(i0 + 3 < NCHUNK)
        def _():
            _stage(i0 + 3, 1)
        return 0

    lax.fori_loop(0, NCHUNK // 2, _pair, 0)
    pltpu.make_async_copy(mv[0], aggr.at[dstall.at[0]], sem_s[0]).wait()
    pltpu.make_async_copy(mv[1], aggr.at[dstall.at[0]], sem_s[1]).wait()
    plsc.subcore_barrier()
    pltpu.sync_copy(aggr.at[pl.ds(s * ZROWS, ZROWS)],
                    out_hbm.at[pl.ds(c * NP_ + s * ZROWS, ZROWS)])


_sc_edge = functools.partial(
    pl.kernel,
    mesh=plsc.VectorSubcoreMesh(core_axis_name="c", subcore_axis_name="s",
                                num_cores=NC, num_subcores=NS),
    out_type=jax.ShapeDtypeStruct((NC * NP_, OCP), jnp.float32),
    compiler_params=pltpu.CompilerParams(needs_layout_passes=False,
                                         use_tc_tiling_on_sc=False,
                                         disable_bounds_checks=True),
    scratch_types=[
        pltpu.VMEM((NCHUNK, CH), jnp.int32),
        pltpu.VMEM((NCHUNK, CH), jnp.int32),
        [pltpu.VMEM((CH, 64), jnp.float32)] * 2,
        [pltpu.VMEM((CH, UW), jnp.float32)] * 2,
        [pltpu.VMEM((CH, OCP), jnp.float32)] * 2,
        pltpu.VMEM((ZROWS, OCP), jnp.float32),
        pltpu.VMEM_SHARED((NP_, OCP), jnp.float32),
        [pltpu.SemaphoreType.DMA] * 2,
        [pltpu.SemaphoreType.DMA] * 2,
        [pltpu.SemaphoreType.DMA] * 2,
    ],
)(_sc_edge_body)


def _mk_mflat(Wn, bn):
    m = jnp.concatenate(
        [Wn.reshape(EF, F, OC).transpose(1, 0, 2),
         bn.reshape(1, F, OC).transpose(1, 0, 2)], axis=1)
    m = m.reshape(F, KC * OC)
    return jnp.pad(m, ((0, 0), (0, UW - KC * OC)))


def _layer(h_pad, src_p, dst_p, a_p, mflat, w1a, w1b, ba1, wa2, ba2):
    u = _tc_u(h_pad, mflat)
    partials = _sc_edge(u, src_p, dst_p, a_p).reshape(NC, NP_, OCP)
    return _tc_update(partials, h_pad, w1a, w1b, ba1, wa2, ba2)


def kernel(x, edge_ind, edge_attr, Wn1, bn1, Wa11, ba11, Wa21, ba21,
           Wn2, bn2, Wa12, ba12, Wa22, ba22, Wmu, bmu, Wlv, blv):
    f32 = jnp.float32
    h0 = jnp.pad(x, ((0, NP_ - N), (0, 0)))
    src_p = jnp.pad(edge_ind[0], (0, EP - E)).reshape(NW, NCHUNK, CH)
    dst_p = jnp.pad(edge_ind[1], (0, EP - E),
                    constant_values=PAD_DST).reshape(NW, NCHUNK, CH)
    a_p = jnp.pad(edge_attr, ((0, EP - E), (0, 64 - EF)))

    mflat1 = _mk_mflat(Wn1, bn1)
    mflat2 = _mk_mflat(Wn2, bn2)
    w1a1 = jnp.pad(Wa11[:OC], ((0, OCP - OC), (0, 0)))
    w1b1 = Wa11[OC:]
    w1a2 = jnp.pad(Wa12[:OC], ((0, OCP - OC), (0, 0)))
    w1b2 = Wa12[OC:]

    h1 = _layer(h0, src_p, dst_p, a_p, mflat1, w1a1, w1b1,
                ba11.reshape(1, F), Wa21, ba21.reshape(1, F))
    h2 = _layer(h1, src_p, dst_p, a_p, mflat2, w1a2, w1b2,
                ba12.reshape(1, F), Wa22, ba22.reshape(1, F))

    whead = jnp.concatenate(
        [jnp.pad(Wmu, ((0, 0), (0, 256 - Z))),
         jnp.pad(Wlv, ((0, 0), (0, 256 - Z)))], axis=1)
    bhead = jnp.concatenate(
        [jnp.pad(bmu, (0, 256 - Z)), jnp.pad(blv, (0, 256 - Z))]
    ).reshape(1, 512)
    heads = _tc_heads(h2, whead, bhead)
    return (heads[:N, :Z], heads[:N, 256:256 + Z])


# speedup vs baseline: 1.2476x; 1.0128x over previous
import functools

import jax
import jax.numpy as jnp
from jax import lax
from jax.experimental import pallas as pl
from jax.experimental.pallas import tpu as pltpu
from jax.experimental.pallas import tpu_sc as plsc

N = 10000
E = 160000
F = 256
EF = 49
OC = 20
Z = 200

NP_ = 10240
KC = EF + 1
UW = 1024
OCP = 32
NC, NS = 2, 16
NW = NC * NS
CH = 32
NCHUNK = 158
PER_W = CH * NCHUNK
EP = NW * PER_W
PAD_DST = N + 100
ZROWS = NP_ // NS
RB = 1024
GR = NP_ // RB


def _mm_u(x_ref, w_ref, o_ref):
    o_ref[...] = jnp.dot(x_ref[...].astype(jnp.bfloat16),
                         w_ref[...].astype(jnp.bfloat16),
                         preferred_element_type=jnp.float32)


def _tc_u(h_pad, mflat):
    return pl.pallas_call(
        _mm_u,
        grid=(GR,),
        in_specs=[
            pl.BlockSpec((RB, F), lambda i: (i, 0)),
            pl.BlockSpec((F, UW), lambda i: (0, 0)),
        ],
        out_specs=pl.BlockSpec((RB, UW), lambda i: (i, 0)),
        out_shape=jax.ShapeDtypeStruct((NP_, UW), jnp.float32),
    )(h_pad, mflat)


def _upd(p_ref, h_ref, w1a_ref, w1b_ref, ba1_ref, wa2_ref, ba2_ref):
    agg = p_ref[0] + p_ref[1]
    y = jnp.dot(agg, w1a_ref[...], preferred_element_type=jnp.float32)
    y = y + jnp.dot(h_ref[...], w1b_ref[...], preferred_element_type=jnp.float32)
    y = jnp.maximum(y + ba1_ref[...], 0.0)
    return jnp.dot(y, wa2_ref[...],
                   preferred_element_type=jnp.float32) + ba2_ref[...]


def _mm_upd_u(p_ref, h_ref, w1a_ref, w1b_ref, ba1_ref, wa2_ref, ba2_ref,
              mf_ref, h1_ref, u_ref):
    h1 = _upd(p_ref, h_ref, w1a_ref, w1b_ref, ba1_ref, wa2_ref, ba2_ref)
    h1_ref[...] = h1
    u_ref[...] = jnp.dot(h1, mf_ref[...], preferred_element_type=jnp.float32)


def _tc_update_u(partials, h_pad, w1a, w1b, ba1, wa2, ba2, mflat2):
    return pl.pallas_call(
        _mm_upd_u,
        grid=(GR,),
        in_specs=[
            pl.BlockSpec((2, RB, OCP), lambda i: (0, i, 0)),
            pl.BlockSpec((RB, F), lambda i: (i, 0)),
            pl.BlockSpec((OCP, F), lambda i: (0, 0)),
            pl.BlockSpec((F, F), lambda i: (0, 0)),
            pl.BlockSpec((1, F), lambda i: (0, 0)),
            pl.BlockSpec((F, F), lambda i: (0, 0)),
            pl.BlockSpec((1, F), lambda i: (0, 0)),
            pl.BlockSpec((F, UW), lambda i: (0, 0)),
        ],
        out_specs=[
            pl.BlockSpec((RB, F), lambda i: (i, 0)),
            pl.BlockSpec((RB, UW), lambda i: (i, 0)),
        ],
        out_shape=[
            jax.ShapeDtypeStruct((NP_, F), jnp.float32),
            jax.ShapeDtypeStruct((NP_, UW), jnp.float32),
        ],
    )(partials, h_pad, w1a, w1b, ba1, wa2, ba2, mflat2)


def _mm_upd_heads(p_ref, h_ref, w1a_ref, w1b_ref, ba1_ref, wa2_ref, ba2_ref,
                  wh_ref, bh_ref, o_ref):
    h2 = _upd(p_ref, h_ref, w1a_ref, w1b_ref, ba1_ref, wa2_ref, ba2_ref)
    o_ref[...] = jnp.dot(h2, wh_ref[...],
                         preferred_element_type=jnp.float32) + bh_ref[...]


def _tc_update_heads(partials, h_pad, w1a, w1b, ba1, wa2, ba2, whead, bhead):
    return pl.pallas_call(
        _mm_upd_heads,
        grid=(GR,),
        in_specs=[
            pl.BlockSpec((2, RB, OCP), lambda i: (0, i, 0)),
            pl.BlockSpec((RB, F), lambda i: (i, 0)),
            pl.BlockSpec((OCP, F), lambda i: (0, 0)),
            pl.BlockSpec((F, F), lambda i: (0, 0)),
            pl.BlockSpec((1, F), lambda i: (0, 0)),
            pl.BlockSpec((F, F), lambda i: (0, 0)),
            pl.BlockSpec((1, F), lambda i: (0, 0)),
            pl.BlockSpec((F, 512), lambda i: (0, 0)),
            pl.BlockSpec((1, 512), lambda i: (0, 0)),
        ],
        out_specs=pl.BlockSpec((RB, 512), lambda i: (i, 0)),
        out_shape=jax.ShapeDtypeStruct((NP_, 512), jnp.float32),
    )(partials, h_pad, w1a, w1b, ba1, wa2, ba2, whead, bhead)


def _sc_edge_body(u_hbm, src_hbm, dst_hbm, a_hbm, out_hbm,
                  srcall, dstall, av, uv, mv, zbuf, aggr,
                  sem_u, sem_a, sem_s):
    c = lax.axis_index("c")
    s = lax.axis_index("s")
    wid = c * NS + s
    iota16 = lax.iota(jnp.int32, 16)
    zero16 = jnp.zeros((16,), jnp.float32)

    for b in range(2):
        for g in range(CH // 16):
            eids0 = iota16 + g * 16
            for o in range(OC, OCP):
                plsc.store_scatter(mv[b],
                                   [eids0, jnp.full((16,), o, jnp.int32)],
                                   zero16)

    pltpu.sync_copy(src_hbm.at[wid], srcall)
    pltpu.sync_copy(dst_hbm.at[wid], dstall)

    def _zb(i, _):
        n = i * 16 + iota16
        plsc.store_scatter(zbuf, [n // OCP, n % OCP], zero16)
        return 0
    lax.fori_loop(0, (ZROWS * OCP) // 16, _zb, 0)
    pltpu.sync_copy(zbuf, aggr.at[pl.ds(s * ZROWS, ZROWS)])
    plsc.subcore_barrier()

    ebase = wid * PER_W

    def _stage(ci, b):
        pltpu.async_copy(a_hbm.at[pl.ds(ebase + ci * CH, CH)], av[b],
                         sem_a[b])
        pltpu.async_copy(u_hbm.at[srcall.at[ci]], uv[b], sem_u[b])

    def _compute(ci, b):
        pltpu.make_async_copy(u_hbm.at[srcall.at[ci]], uv[b], sem_u[b]).wait()
        pltpu.make_async_copy(a_hbm.at[pl.ds(ebase, CH)], av[b],
                              sem_a[b]).wait()

        @pl.when(ci >= 2)
        def _():
            pltpu.make_async_copy(mv[b], aggr.at[dstall.at[ci]],
                                  sem_s[b]).wait()

        def _edge4(i, _):
            rows = tuple(jnp.full((16,), i * 4 + j, jnp.int32)
                         for j in range(4))
            acc0 = [plsc.load_gather(uv[b], [rows[j], EF * OC + iota16])
                    for j in range(4)]
            acc1 = [plsc.load_gather(uv[b], [rows[j], EF * OC + 16 + iota16])
                    for j in range(4)]

            def _k(k, carry):
                acc0, acc1, u0c, u1c, ac = carry
                for j in range(4):
                    a_k = plsc.load_gather(av[b], [rows[j], ac])
                    acc0[j] = acc0[j] + a_k * plsc.load_gather(
                        uv[b], [rows[j], u0c])
                    acc1[j] = acc1[j] + a_k * plsc.load_gather(
                        uv[b], [rows[j], u1c])
                return acc0, acc1, u0c + OC, u1c + OC, ac + 1

            acc0, acc1, _, _, _ = lax.fori_loop(
                0, EF, _k,
                (acc0, acc1, iota16, 16 + iota16,
                 jnp.zeros((16,), jnp.int32)))
            for j in range(4):
                m0 = 1.0 / (1.0 + jnp.exp(-acc0[j]))
                m1 = 1.0 / (1.0 + jnp.exp(-acc1[j]))
                plsc.store_scatter(mv[b], [rows[j], iota16], m0)
                plsc.store_scatter(mv[b], [rows[j], 16 + iota16], m1)
            return 0

        lax.fori_loop(0, CH // 4, _edge4, 0)
        pltpu.async_copy(mv[b], aggr.at[dstall.at[ci]], sem_s[b], add=True)

    _stage(0, 0)
    _stage(1, 1)

    def _pair(j, _):
        i0 = 2 * j
        _compute(i0, 0)

        @pl.when(i0 + 2 < NCHUNK)
        def _():
            _stage(i0 + 2, 0)
        _compute(i0 + 1, 1)

        @pl.when(i0 + 3 < NCHUNK)
        def _():
            _stage(i0 + 3, 1)
        return 0

    lax.fori_loop(0, NCHUNK // 2, _pair, 0)
    pltpu.make_async_copy(mv[0], aggr.at[dstall.at[0]], sem_s[0]).wait()
    pltpu.make_async_copy(mv[1], aggr.at[dstall.at[0]], sem_s[1]).wait()
    plsc.subcore_barrier()
    pltpu.sync_copy(aggr.at[pl.ds(s * ZROWS, ZROWS)],
                    out_hbm.at[pl.ds(c * NP_ + s * ZROWS, ZROWS)])


_sc_edge = functools.partial(
    pl.kernel,
    mesh=plsc.VectorSubcoreMesh(core_axis_name="c", subcore_axis_name="s",
                                num_cores=NC, num_subcores=NS),
    out_type=jax.ShapeDtypeStruct((NC * NP_, OCP), jnp.float32),
    compiler_params=pltpu.CompilerParams(needs_layout_passes=False,
                                         use_tc_tiling_on_sc=False,
                                         disable_bounds_checks=True),
    scratch_types=[
        pltpu.VMEM((NCHUNK, CH), jnp.int32),
        pltpu.VMEM((NCHUNK, CH), jnp.int32),
        [pltpu.VMEM((CH, 64), jnp.float32)] * 2,
        [pltpu.VMEM((CH, UW), jnp.float32)] * 2,
        [pltpu.VMEM((CH, OCP), jnp.float32)] * 2,
        pltpu.VMEM((ZROWS, OCP), jnp.float32),
        pltpu.VMEM_SHARED((NP_, OCP), jnp.float32),
        [pltpu.SemaphoreType.DMA] * 2,
        [pltpu.SemaphoreType.DMA] * 2,
        [pltpu.SemaphoreType.DMA] * 2,
    ],
)(_sc_edge_body)


def _mk_mflat(Wn, bn):
    m = jnp.concatenate(
        [Wn.reshape(EF, F, OC).transpose(1, 0, 2),
         bn.reshape(1, F, OC).transpose(1, 0, 2)], axis=1)
    m = m.reshape(F, KC * OC)
    return jnp.pad(m, ((0, 0), (0, UW - KC * OC)))


def _run_sc(u, src_p, dst_p, a_p):
    return _sc_edge(u, src_p, dst_p, a_p).reshape(NC, NP_, OCP)


def kernel(x, edge_ind, edge_attr, Wn1, bn1, Wa11, ba11, Wa21, ba21,
           Wn2, bn2, Wa12, ba12, Wa22, ba22, Wmu, bmu, Wlv, blv):
    f32 = jnp.float32
    h0 = jnp.pad(x, ((0, NP_ - N), (0, 0)))
    src_p = jnp.pad(edge_ind[0], (0, EP - E)).reshape(NW, NCHUNK, CH)
    dst_p = jnp.pad(edge_ind[1], (0, EP - E),
                    constant_values=PAD_DST).reshape(NW, NCHUNK, CH)
    a_p = jnp.pad(edge_attr, ((0, EP - E), (0, 64 - EF)))

    mflat1 = _mk_mflat(Wn1, bn1)
    mflat2 = _mk_mflat(Wn2, bn2)
    w1a1 = jnp.pad(Wa11[:OC], ((0, OCP - OC), (0, 0)))
    w1b1 = Wa11[OC:]
    w1a2 = jnp.pad(Wa12[:OC], ((0, OCP - OC), (0, 0)))
    w1b2 = Wa12[OC:]

    whead = jnp.concatenate(
        [jnp.pad(Wmu, ((0, 0), (0, 256 - Z))),
         jnp.pad(Wlv, ((0, 0), (0, 256 - Z)))], axis=1)
    bhead = jnp.concatenate(
        [jnp.pad(bmu, (0, 256 - Z)), jnp.pad(blv, (0, 256 - Z))]
    ).reshape(1, 512)

    u1 = _tc_u(h0, mflat1)
    p1 = _run_sc(u1, src_p, dst_p, a_p)
    h1, u2 = _tc_update_u(p1, h0, w1a1, w1b1, ba11.reshape(1, F),
                          Wa21, ba21.reshape(1, F), mflat2)
    p2 = _run_sc(u2, src_p, dst_p, a_p)
    heads = _tc_update_heads(p2, h1, w1a2, w1b2, ba12.reshape(1, F),
                             Wa22, ba22.reshape(1, F), whead, bhead)
    return (heads[:N, :Z], heads[:N, 256:256 + Z])
